# baseline (speedup 1.0000x reference)
import functools

import jax
import jax.numpy as jnp
from jax import lax
from jax.experimental import pallas as pl
from jax.experimental.pallas import tpu as pltpu
from jax.experimental.pallas import tpu_sc as plsc

_D = 1024
_VOCAB = 16000
_B = 4
_S = 2048
_L = 16
_NSL = _D // _L
_NW = 32
_S_PER_W = _S // _NW
_RCH = 32
_EPS = 1e-5


def _rsqrt(x):
    i = lax.bitcast_convert_type(x, jnp.int32)
    i = jnp.int32(0x5F3759DF) - lax.shift_right_arithmetic(i, jnp.int32(1))
    y = lax.bitcast_convert_type(i, jnp.float32)
    for _ in range(3):
        y = y * (1.5 - 0.5 * x * y * y)
    return y


_GDN = lax.GatherDimensionNumbers(
    offset_dims=(), collapsed_slice_dims=(0,), start_index_map=(0,))


def _lane_sum(v):
    idx = lax.iota(jnp.int32, _L)
    for sh in (8, 4, 2, 1):
        perm = lax.bitwise_xor(idx, jnp.int32(sh))
        sw = lax.gather(v, perm[:, None], _GDN, slice_sizes=(1,),
                        mode=lax.GatherScatterMode.PROMISE_IN_BOUNDS)
        v = v + sw
    return v[0]


def _sc_body(ids_hbm, mask_hbm, tok_hbm, pos_hbm, gam_hbm, bet_hbm, out_hbm,
             idx_v, mask_v, tok_v, pos_v, gam_v, bet_v, sem):
    cid = lax.axis_index("c")
    sid = lax.axis_index("s")
    wid = sid * 2 + cid
    s_base = wid * _S_PER_W

    pltpu.sync_copy(gam_hbm, gam_v)
    pltpu.sync_copy(bet_hbm, bet_v)

    for sc in range(_S_PER_W // _RCH):
        s0 = s_base + sc * _RCH
        pltpu.sync_copy(pos_hbm.at[pl.ds(s0, _RCH)], pos_v)
        for b in range(_B):
            base = b * _S + s0
            pltpu.sync_copy(ids_hbm.at[pl.ds(base, _RCH)], idx_v)
            pltpu.sync_copy(mask_hbm.at[pl.ds(base, _RCH)], mask_v.at[pl.ds(0, _RCH)])
            pltpu.async_copy(tok_hbm.at[idx_v], tok_v, sem).wait()

            def row_body(r, _):
                def sl1(j, carry):
                    s_acc, q_acc = carry
                    off = pl.multiple_of(j * _L, _L)
                    v = tok_v[r, pl.ds(off, _L)] + pos_v[r, pl.ds(off, _L)]
                    tok_v[r, pl.ds(off, _L)] = v
                    return (s_acc + v, q_acc + v * v)

                s_acc, q_acc = lax.fori_loop(
                    0, _NSL, sl1,
                    (jnp.zeros((_L,), jnp.float32), jnp.zeros((_L,), jnp.float32)))
                total = _lane_sum(s_acc)
                sq = _lane_sum(q_acc)
                mean = total * (1.0 / _D)
                var = sq * (1.0 / _D) - mean * mean
                rstd = _rsqrt(var + _EPS)
                m = mask_v[pl.ds(r, _L)][0]

                def sl2(j, _):
                    off = pl.multiple_of(j * _L, _L)
                    x = tok_v[r, pl.ds(off, _L)]
                    y = (x - mean) * rstd * gam_v[pl.ds(off, _L)] + bet_v[pl.ds(off, _L)]
                    tok_v[r, pl.ds(off, _L)] = y * m
                    return 0

                lax.fori_loop(0, _NSL, sl2, 0)
                return 0

            lax.fori_loop(0, _RCH, row_body, 0)
            pltpu.sync_copy(tok_v, out_hbm.at[pl.ds(base, _RCH)])


@functools.partial(
    pl.kernel,
    out_type=jax.ShapeDtypeStruct((_B * _S, _D), jnp.float32),
    mesh=plsc.VectorSubcoreMesh(core_axis_name="c", subcore_axis_name="s"),
    scratch_types=[
        pltpu.VMEM((_RCH,), jnp.int32),
        pltpu.VMEM((_RCH + _L,), jnp.float32),
        pltpu.VMEM((_RCH, _D), jnp.float32),
        pltpu.VMEM((_RCH, _D), jnp.float32),
        pltpu.VMEM((_D,), jnp.float32),
        pltpu.VMEM((_D,), jnp.float32),
        pltpu.SemaphoreType.DMA,
    ],
)
def _sc_embed_ln(ids_hbm, mask_hbm, tok_hbm, pos_hbm, gam_hbm, bet_hbm, out_hbm,
                 idx_v, mask_v, tok_v, pos_v, gam_v, bet_v, sem):
    _sc_body(ids_hbm, mask_hbm, tok_hbm, pos_hbm, gam_hbm, bet_hbm, out_hbm,
             idx_v, mask_v, tok_v, pos_v, gam_v, bet_v, sem)


@jax.jit
def kernel(input_ids, attention_mask, token_emb, pos_emb, ln_gamma, ln_beta):
    ids = input_ids.reshape(-1).astype(jnp.int32)
    msk = attention_mask.reshape(-1).astype(jnp.float32)
    out = _sc_embed_ln(ids, msk, token_emb, pos_emb, ln_gamma, ln_beta)
    return out.reshape(_B, _S, _D)


# speedup vs baseline: 3.0137x; 3.0137x over previous
import functools

import jax
import jax.numpy as jnp
from jax import lax
from jax.experimental import pallas as pl
from jax.experimental.pallas import tpu as pltpu
from jax.experimental.pallas import tpu_sc as plsc

_D = 1024
_B = 4
_S = 2048
_N = _B * _S
_NW = 32
_RPW = _N // _NW
_RCH = 32
_NCH = _RPW // _RCH
_BLK = 128
_EPS = 1e-5


def _sc_gather_body(ids_hbm, tok_hbm, out_hbm, idx_v, buf0, buf1, sem0, sem1):
    cid = lax.axis_index("c")
    sid = lax.axis_index("s")
    wid = sid * 2 + cid
    base = wid * _RPW

    pltpu.sync_copy(ids_hbm.at[pl.ds(base, _RPW)], idx_v)

    bufs = (buf0, buf1)
    sems = (sem0, sem1)
    handles = [None, None]
    handles[0] = pltpu.async_copy(
        tok_hbm.at[idx_v.at[pl.ds(0, _RCH)]], bufs[0], sems[0])
    for k in range(_NCH):
        if k + 1 < _NCH:
            handles[(k + 1) % 2] = pltpu.async_copy(
                tok_hbm.at[idx_v.at[pl.ds((k + 1) * _RCH, _RCH)]],
                bufs[(k + 1) % 2], sems[(k + 1) % 2])
        handles[k % 2].wait()
        pltpu.sync_copy(bufs[k % 2], out_hbm.at[pl.ds(base + k * _RCH, _RCH)])


@functools.partial(
    pl.kernel,
    out_type=jax.ShapeDtypeStruct((_N, _D), jnp.float32),
    mesh=plsc.VectorSubcoreMesh(core_axis_name="c", subcore_axis_name="s"),
    scratch_types=[
        pltpu.VMEM((_RPW,), jnp.int32),
        pltpu.VMEM((_RCH, _D), jnp.float32),
        pltpu.VMEM((_RCH, _D), jnp.float32),
        pltpu.SemaphoreType.DMA,
        pltpu.SemaphoreType.DMA,
    ],
)
def _sc_gather(ids_hbm, tok_hbm, out_hbm, idx_v, buf0, buf1, sem0, sem1):
    _sc_gather_body(ids_hbm, tok_hbm, out_hbm, idx_v, buf0, buf1, sem0, sem1)


def _ln_body(x_ref, pos_ref, msk_ref, gam_ref, bet_ref, out_ref):
    x = x_ref[...] + pos_ref[...]
    mean = jnp.mean(x, axis=-1, keepdims=True)
    var = jnp.mean(x * x, axis=-1, keepdims=True) - mean * mean
    y = (x - mean) * lax.rsqrt(var + _EPS) * gam_ref[...] + bet_ref[...]
    m = jnp.reshape(msk_ref[0, 0, :], (_BLK, 1))
    out_ref[...] = y * m


def _tc_layernorm(gathered, pos_emb, mask3, gamma2, beta2):
    n_sblk = _S // _BLK
    return pl.pallas_call(
        _ln_body,
        grid=(n_sblk, _B),
        in_specs=[
            pl.BlockSpec((_BLK, _D), lambda i, j: (j * n_sblk + i, 0)),
            pl.BlockSpec((_BLK, _D), lambda i, j: (i, 0)),
            pl.BlockSpec((1, 1, _BLK), lambda i, j: (j * n_sblk + i, 0, 0)),
            pl.BlockSpec((1, _D), lambda i, j: (0, 0)),
            pl.BlockSpec((1, _D), lambda i, j: (0, 0)),
        ],
        out_specs=pl.BlockSpec((_BLK, _D), lambda i, j: (j * n_sblk + i, 0)),
        out_shape=jax.ShapeDtypeStruct((_N, _D), jnp.float32),
    )(gathered, pos_emb, mask3, gamma2, beta2)


@jax.jit
def kernel(input_ids, attention_mask, token_emb, pos_emb, ln_gamma, ln_beta):
    ids = input_ids.reshape(-1).astype(jnp.int32)
    gathered = _sc_gather(ids, token_emb)
    mask3 = attention_mask.astype(jnp.float32).reshape(_N // _BLK, 1, _BLK)
    out = _tc_layernorm(gathered, pos_emb,
                        mask3, ln_gamma.reshape(1, _D), ln_beta.reshape(1, _D))
    return out.reshape(_B, _S, _D)
